# baseline (speedup 1.0000x reference)
import jax
import jax.numpy as jnp
from jax.experimental import pallas as pl

_N = 10000
_F_IN = 128
_F_OUT = 32
_BLOCK = 1000


def _body(x_ref, wi_ref, wc_ref, wo_ref, bi_ref, bc_ref, bo_ref,
          wco_ref, wlin_ref, blin_ref, out_ref):
    x = x_ref[:]
    gi = jnp.dot(x, wi_ref[:], preferred_element_type=jnp.float32) + bi_ref[:]
    gc = jnp.dot(x, wc_ref[:], preferred_element_type=jnp.float32) + bc_ref[:]
    go = jnp.dot(x, wo_ref[:], preferred_element_type=jnp.float32) + bo_ref[:]
    C = jax.nn.sigmoid(gi) * jnp.tanh(gc)
    O = jax.nn.sigmoid(go + wco_ref[:] * C)
    h = jnp.maximum(O * jnp.tanh(C), 0.0)
    out_ref[:] = jnp.sum(h * wlin_ref[:], axis=1, keepdims=True) + blin_ref[:]


def kernel(x, edge_index, edge_weight,
           W_x_i, b_x_i, W_h_i, b_h_i, b_i,
           W_x_f, b_x_f, W_h_f, b_h_f, b_f,
           W_x_c, b_x_c, W_h_c, b_h_c, b_c,
           W_x_o, b_x_o, W_h_o, b_h_o, b_o,
           w_c_i, w_c_f, w_c_o, W_lin, b_lin):
    del edge_index, edge_weight, W_h_i, W_h_f, W_h_c, W_h_o
    del W_x_f, b_x_f, b_h_f, b_f, w_c_i, w_c_f
    bi = (b_x_i + b_h_i + b_i).reshape(1, _F_OUT)
    bc = (b_x_c + b_h_c + b_c).reshape(1, _F_OUT)
    bo = (b_x_o + b_h_o + b_o).reshape(1, _F_OUT)
    wlin_row = W_lin.reshape(1, _F_OUT)
    blin = b_lin.reshape(1, 1)

    rep = lambda shape: pl.BlockSpec(shape, lambda i: (0, 0))
    return pl.pallas_call(
        _body,
        grid=(_N // _BLOCK,),
        in_specs=[
            pl.BlockSpec((_BLOCK, _F_IN), lambda i: (i, 0)),
            rep((_F_IN, _F_OUT)), rep((_F_IN, _F_OUT)), rep((_F_IN, _F_OUT)),
            rep((1, _F_OUT)), rep((1, _F_OUT)), rep((1, _F_OUT)),
            rep((1, _F_OUT)), rep((1, _F_OUT)), rep((1, 1)),
        ],
        out_specs=pl.BlockSpec((_BLOCK, 1), lambda i: (i, 0)),
        out_shape=jax.ShapeDtypeStruct((_N, 1), jnp.float32),
    )(x, W_x_i, W_x_c, W_x_o, bi, bc, bo, w_c_o, wlin_row, blin)


# speedup vs baseline: 1.1177x; 1.1177x over previous
import jax
import jax.numpy as jnp
from jax.experimental import pallas as pl

_N = 10000
_F_IN = 128
_F_OUT = 32
_BLOCK = 2000


def _body(x_ref, wi_ref, wc_ref, wo_ref, bi_ref, bc_ref, bo_ref,
          wco_ref, wlin_ref, blin_ref, out_ref):
    x = x_ref[:]
    gi = jnp.dot(x, wi_ref[:], preferred_element_type=jnp.float32) + bi_ref[:]
    gc = jnp.dot(x, wc_ref[:], preferred_element_type=jnp.float32) + bc_ref[:]
    go = jnp.dot(x, wo_ref[:], preferred_element_type=jnp.float32) + bo_ref[:]
    C = jax.nn.sigmoid(gi) * jnp.tanh(gc)
    O = jax.nn.sigmoid(go + wco_ref[:] * C)
    h = jnp.maximum(O * jnp.tanh(C), 0.0)
    out_ref[:] = jnp.sum(h * wlin_ref[:], axis=1, keepdims=True) + blin_ref[:]


def kernel(x, edge_index, edge_weight,
           W_x_i, b_x_i, W_h_i, b_h_i, b_i,
           W_x_f, b_x_f, W_h_f, b_h_f, b_f,
           W_x_c, b_x_c, W_h_c, b_h_c, b_c,
           W_x_o, b_x_o, W_h_o, b_h_o, b_o,
           w_c_i, w_c_f, w_c_o, W_lin, b_lin):
    del edge_index, edge_weight, W_h_i, W_h_f, W_h_c, W_h_o
    del W_x_f, b_x_f, b_h_f, b_f, w_c_i, w_c_f
    bi = (b_x_i + b_h_i + b_i).reshape(1, _F_OUT)
    bc = (b_x_c + b_h_c + b_c).reshape(1, _F_OUT)
    bo = (b_x_o + b_h_o + b_o).reshape(1, _F_OUT)
    wlin_row = W_lin.reshape(1, _F_OUT)
    blin = b_lin.reshape(1, 1)

    rep = lambda shape: pl.BlockSpec(shape, lambda i: (0, 0))
    return pl.pallas_call(
        _body,
        grid=(_N // _BLOCK,),
        in_specs=[
            pl.BlockSpec((_BLOCK, _F_IN), lambda i: (i, 0)),
            rep((_F_IN, _F_OUT)), rep((_F_IN, _F_OUT)), rep((_F_IN, _F_OUT)),
            rep((1, _F_OUT)), rep((1, _F_OUT)), rep((1, _F_OUT)),
            rep((1, _F_OUT)), rep((1, _F_OUT)), rep((1, 1)),
        ],
        out_specs=pl.BlockSpec((_BLOCK, 1), lambda i: (i, 0)),
        out_shape=jax.ShapeDtypeStruct((_N, 1), jnp.float32),
    )(x, W_x_i, W_x_c, W_x_o, bi, bc, bo, w_c_o, wlin_row, blin)


# speedup vs baseline: 1.1188x; 1.0010x over previous
import jax
import jax.numpy as jnp
from jax.experimental import pallas as pl

_N = 10000
_F_IN = 128
_F_OUT = 32
_BLOCK = 10000


def _body(x_ref, wi_ref, wc_ref, wo_ref, bi_ref, bc_ref, bo_ref,
          wco_ref, wlin_ref, blin_ref, out_ref):
    x = x_ref[:]
    gi = jnp.dot(x, wi_ref[:], preferred_element_type=jnp.float32) + bi_ref[:]
    gc = jnp.dot(x, wc_ref[:], preferred_element_type=jnp.float32) + bc_ref[:]
    go = jnp.dot(x, wo_ref[:], preferred_element_type=jnp.float32) + bo_ref[:]
    C = jax.nn.sigmoid(gi) * jnp.tanh(gc)
    O = jax.nn.sigmoid(go + wco_ref[:] * C)
    h = jnp.maximum(O * jnp.tanh(C), 0.0)
    out_ref[:] = jnp.sum(h * wlin_ref[:], axis=1, keepdims=True) + blin_ref[:]


def kernel(x, edge_index, edge_weight,
           W_x_i, b_x_i, W_h_i, b_h_i, b_i,
           W_x_f, b_x_f, W_h_f, b_h_f, b_f,
           W_x_c, b_x_c, W_h_c, b_h_c, b_c,
           W_x_o, b_x_o, W_h_o, b_h_o, b_o,
           w_c_i, w_c_f, w_c_o, W_lin, b_lin):
    del edge_index, edge_weight, W_h_i, W_h_f, W_h_c, W_h_o
    del W_x_f, b_x_f, b_h_f, b_f, w_c_i, w_c_f
    bi = (b_x_i + b_h_i + b_i).reshape(1, _F_OUT)
    bc = (b_x_c + b_h_c + b_c).reshape(1, _F_OUT)
    bo = (b_x_o + b_h_o + b_o).reshape(1, _F_OUT)
    wlin_row = W_lin.reshape(1, _F_OUT)
    blin = b_lin.reshape(1, 1)

    rep = lambda shape: pl.BlockSpec(shape, lambda i: (0, 0))
    return pl.pallas_call(
        _body,
        grid=(_N // _BLOCK,),
        in_specs=[
            pl.BlockSpec((_BLOCK, _F_IN), lambda i: (i, 0)),
            rep((_F_IN, _F_OUT)), rep((_F_IN, _F_OUT)), rep((_F_IN, _F_OUT)),
            rep((1, _F_OUT)), rep((1, _F_OUT)), rep((1, _F_OUT)),
            rep((1, _F_OUT)), rep((1, _F_OUT)), rep((1, 1)),
        ],
        out_specs=pl.BlockSpec((_BLOCK, 1), lambda i: (i, 0)),
        out_shape=jax.ShapeDtypeStruct((_N, 1), jnp.float32),
    )(x, W_x_i, W_x_c, W_x_o, bi, bc, bo, w_c_o, wlin_row, blin)
